# baseline (speedup 1.0000x reference)
import functools

import jax
import jax.numpy as jnp
from jax import lax
from jax.experimental import pallas as pl
from jax.experimental.pallas import tpu as pltpu
from jax.experimental.pallas import tpu_sc as plsc

B, D, H, W = 4, 96, 96, 96
N = D * H * W
TOT = B * N
NTILES = 32
SPT = TOT // NTILES
CHUNK = 1536
NV = CHUNK // 16
NC = SPT // CHUNK
FMAX = float(W - 1)
SHIFTS = (0, 1, W, W + 1, H * W, H * W + 1, H * W + W, H * W + W + 1)


def _coords(transfos):
    zs = jnp.linspace(-1.0, 1.0, D)
    ys = jnp.linspace(-1.0, 1.0, H)
    xs = jnp.linspace(-1.0, 1.0, W)
    mz, my, mx = jnp.meshgrid(zs, ys, xs, indexing="ij")
    grid = jnp.stack([mx.reshape(-1), my.reshape(-1), mz.reshape(-1),
                      jnp.ones(N, dtype=jnp.float32)], axis=0)
    points = jnp.einsum("bij,jn->bin", transfos, grid)
    coeff = jnp.float32((W - 1) / 2.0)
    ix = coeff * points[:, 0, :] + coeff
    iy = coeff * points[:, 1, :] + coeff
    iz = coeff * points[:, 2, :] + coeff
    return ix.reshape(-1), iy.reshape(-1), iz.reshape(-1)


@functools.partial(
    pl.kernel,
    out_type=jax.ShapeDtypeStruct((TOT,), jnp.float32),
    mesh=plsc.VectorSubcoreMesh(core_axis_name="c", subcore_axis_name="s"),
    compiler_params=pltpu.CompilerParams(needs_layout_passes=False,
                                         use_tc_tiling_on_sc=False),
    scratch_types=[
        pltpu.VMEM((3 * CHUNK,), jnp.float32),
        pltpu.VMEM((CHUNK,), jnp.int32),
        pltpu.VMEM((8 * CHUNK,), jnp.float32),
        pltpu.VMEM((CHUNK, 8), jnp.float32),
        pltpu.VMEM((CHUNK,), jnp.float32),
        pltpu.SemaphoreType.DMA,
    ],
)
def _interp(img2_hbm, ix_hbm, iy_hbm, iz_hbm, out_hbm,
            crd_v, idx_v, w_v, vals_v, out_v, sem):
    wid = lax.axis_index("s") * 2 + lax.axis_index("c")
    bbase = (wid // (NTILES // B)) * N

    def chunk_body(g, carry):
        start = wid * SPT + g * CHUNK
        pltpu.sync_copy(ix_hbm.at[pl.ds(start, CHUNK)], crd_v.at[pl.ds(0, CHUNK)])
        pltpu.sync_copy(iy_hbm.at[pl.ds(start, CHUNK)], crd_v.at[pl.ds(CHUNK, CHUNK)])
        pltpu.sync_copy(iz_hbm.at[pl.ds(start, CHUNK)], crd_v.at[pl.ds(2 * CHUNK, CHUNK)])
        bbv = jnp.full((16,), bbase, jnp.int32)

        def pass1(v, c2):
            base = v * 16
            ix = crd_v[pl.ds(base, 16)]
            iy = crd_v[pl.ds(CHUNK + base, 16)]
            iz = crd_v[pl.ds(2 * CHUNK + base, 16)]
            valid = ((ix >= 0.0) & (ix <= FMAX) & (iy >= 0.0) & (iy <= FMAX)
                     & (iz >= 0.0) & (iz <= FMAX))
            vf = jnp.where(valid, jnp.float32(1.0), jnp.float32(0.0))
            ix0 = jnp.clip(ix, 0.0, FMAX).astype(jnp.int32)
            iy0 = jnp.clip(iy, 0.0, FMAX).astype(jnp.int32)
            iz0 = jnp.clip(iz, 0.0, FMAX).astype(jnp.int32)
            fx = ix - ix0.astype(jnp.float32)
            fy = iy - iy0.astype(jnp.float32)
            fz = iz - iz0.astype(jnp.float32)
            ux = (jnp.abs(fx - 1.0) * vf, jnp.abs(fx) * vf)
            uy = (jnp.abs(fy - 1.0), jnp.abs(fy))
            uz = (jnp.abs(fz - 1.0), jnp.abs(fz))
            idx_v[pl.ds(base, 16)] = (ix0 + iy0 * W) + (iz0 * (H * W) + bbv)
            for c in range(8):
                dx, dy, dz = c >> 2, (c >> 1) & 1, c & 1
                w_v[pl.ds(c * CHUNK + base, 16)] = ux[dx] * (uy[dy] * uz[dz])
            return c2

        lax.fori_loop(0, NV, pass1, 0)
        pltpu.async_copy(img2_hbm.at[idx_v], vals_v, sem).wait()

        def pass2(v, c2):
            base = v * 16
            rows = lax.iota(jnp.int32, 16) + base
            acc = None
            for c in range(8):
                dx, dy, dz = c >> 2, (c >> 1) & 1, c & 1
                col = jnp.full((16,), (dz << 2) | (dy << 1) | dx, jnp.int32)
                vc = plsc.load_gather(vals_v, [rows, col])
                wv = w_v[pl.ds(c * CHUNK + base, 16)]
                acc = wv * vc if acc is None else acc + wv * vc
            out_v[pl.ds(base, 16)] = acc
            return c2

        lax.fori_loop(0, NV, pass2, 0)
        pltpu.sync_copy(out_v, out_hbm.at[pl.ds(start, CHUNK)])
        return carry

    lax.fori_loop(0, NC, chunk_body, 0)


def kernel(img, transfos):
    img_flat = img.reshape(-1)
    img2 = jnp.stack([jnp.roll(img_flat, -s) for s in SHIFTS], axis=1)
    ix, iy, iz = _coords(transfos)
    out = _interp(img2, ix, iy, iz)
    return out.reshape(img.shape)


# speedup vs baseline: 2.2717x; 2.2717x over previous
import functools

import jax
import jax.numpy as jnp
from jax import lax
from jax.experimental import pallas as pl
from jax.experimental.pallas import tpu as pltpu
from jax.experimental.pallas import tpu_sc as plsc

B, D, H, W = 4, 96, 96, 96
N = D * H * W
TOT = B * N
NTILES = 32
SPT = TOT // NTILES
CHUNK = 1536
NV = CHUNK // 16
NC = SPT // CHUNK
FMAX = float(W - 1)
SHIFTS = (0, 1, W, W + 1, H * W, H * W + 1, H * W + W, H * W + W + 1)


def _coords(transfos):
    zs = jnp.linspace(-1.0, 1.0, D)
    ys = jnp.linspace(-1.0, 1.0, H)
    xs = jnp.linspace(-1.0, 1.0, W)
    mz, my, mx = jnp.meshgrid(zs, ys, xs, indexing="ij")
    grid = jnp.stack([mx.reshape(-1), my.reshape(-1), mz.reshape(-1),
                      jnp.ones(N, dtype=jnp.float32)], axis=0)
    points = jnp.einsum("bij,jn->bin", transfos, grid)
    coeff = jnp.float32((W - 1) / 2.0)
    ix = coeff * points[:, 0, :] + coeff
    iy = coeff * points[:, 1, :] + coeff
    iz = coeff * points[:, 2, :] + coeff
    return ix.reshape(-1), iy.reshape(-1), iz.reshape(-1)


PAD = 9320
HHW = 1536 + PAD


@functools.partial(
    pl.kernel,
    out_type=jax.ShapeDtypeStruct((TOT, 8), jnp.float32),
    mesh=plsc.VectorSubcoreMesh(core_axis_name="c", subcore_axis_name="s"),
    compiler_params=pltpu.CompilerParams(needs_layout_passes=False,
                                         use_tc_tiling_on_sc=False),
    scratch_types=[
        pltpu.VMEM((HHW,), jnp.float32),
        pltpu.VMEM((1536, 8), jnp.float32),
    ],
)
def _build(imgpad_hbm, img2_hbm, win_v, buf_v):
    wid = lax.axis_index("s") * 2 + lax.axis_index("c")
    lanes = lax.iota(jnp.int32, 16)

    def chunk_body(g, carry):
        s = wid * SPT + g * 1536
        pltpu.sync_copy(imgpad_hbm.at[pl.ds(s, HHW)], win_v)

        def rows16(v, c2):
            base = v * 16
            rows = lanes + base
            for j, sh in enumerate(SHIFTS):
                vec = plsc.load_gather(win_v, [rows + sh])
                plsc.store_scatter(buf_v, [rows, jnp.full((16,), j, jnp.int32)], vec)
            return c2

        lax.fori_loop(0, NV, rows16, 0)
        pltpu.sync_copy(buf_v, img2_hbm.at[pl.ds(s, 1536)])
        return carry

    lax.fori_loop(0, NC, chunk_body, 0)


@functools.partial(
    pl.kernel,
    out_type=jax.ShapeDtypeStruct((TOT,), jnp.float32),
    mesh=plsc.VectorSubcoreMesh(core_axis_name="c", subcore_axis_name="s"),
    compiler_params=pltpu.CompilerParams(needs_layout_passes=False,
                                         use_tc_tiling_on_sc=False),
    scratch_types=[
        pltpu.VMEM((3 * CHUNK,), jnp.float32),
        pltpu.VMEM((CHUNK,), jnp.int32),
        pltpu.VMEM((8 * CHUNK,), jnp.float32),
        pltpu.VMEM((CHUNK, 8), jnp.float32),
        pltpu.VMEM((CHUNK,), jnp.float32),
        pltpu.SemaphoreType.DMA,
    ],
)
def _interp(img2_hbm, ix_hbm, iy_hbm, iz_hbm, out_hbm,
            crd_v, idx_v, w_v, vals_v, out_v, sem):
    wid = lax.axis_index("s") * 2 + lax.axis_index("c")
    bbase = (wid // (NTILES // B)) * N

    def chunk_body(g, carry):
        start = wid * SPT + g * CHUNK
        pltpu.sync_copy(ix_hbm.at[pl.ds(start, CHUNK)], crd_v.at[pl.ds(0, CHUNK)])
        pltpu.sync_copy(iy_hbm.at[pl.ds(start, CHUNK)], crd_v.at[pl.ds(CHUNK, CHUNK)])
        pltpu.sync_copy(iz_hbm.at[pl.ds(start, CHUNK)], crd_v.at[pl.ds(2 * CHUNK, CHUNK)])
        bbv = jnp.full((16,), bbase, jnp.int32)

        def pass1(v, c2):
            base = v * 16
            ix = crd_v[pl.ds(base, 16)]
            iy = crd_v[pl.ds(CHUNK + base, 16)]
            iz = crd_v[pl.ds(2 * CHUNK + base, 16)]
            valid = ((ix >= 0.0) & (ix <= FMAX) & (iy >= 0.0) & (iy <= FMAX)
                     & (iz >= 0.0) & (iz <= FMAX))
            vf = jnp.where(valid, jnp.float32(1.0), jnp.float32(0.0))
            ix0 = jnp.clip(ix, 0.0, FMAX).astype(jnp.int32)
            iy0 = jnp.clip(iy, 0.0, FMAX).astype(jnp.int32)
            iz0 = jnp.clip(iz, 0.0, FMAX).astype(jnp.int32)
            fx = ix - ix0.astype(jnp.float32)
            fy = iy - iy0.astype(jnp.float32)
            fz = iz - iz0.astype(jnp.float32)
            ux = (jnp.abs(fx - 1.0) * vf, jnp.abs(fx) * vf)
            uy = (jnp.abs(fy - 1.0), jnp.abs(fy))
            uz = (jnp.abs(fz - 1.0), jnp.abs(fz))
            idx_v[pl.ds(base, 16)] = (ix0 + iy0 * W) + (iz0 * (H * W) + bbv)
            for c in range(8):
                dx, dy, dz = c >> 2, (c >> 1) & 1, c & 1
                w_v[pl.ds(c * CHUNK + base, 16)] = ux[dx] * (uy[dy] * uz[dz])
            return c2

        lax.fori_loop(0, NV, pass1, 0)
        pltpu.async_copy(img2_hbm.at[idx_v], vals_v, sem).wait()

        def pass2(v, c2):
            base = v * 16
            rows = lax.iota(jnp.int32, 16) + base
            acc = None
            for c in range(8):
                dx, dy, dz = c >> 2, (c >> 1) & 1, c & 1
                col = jnp.full((16,), (dz << 2) | (dy << 1) | dx, jnp.int32)
                vc = plsc.load_gather(vals_v, [rows, col])
                wv = w_v[pl.ds(c * CHUNK + base, 16)]
                acc = wv * vc if acc is None else acc + wv * vc
            out_v[pl.ds(base, 16)] = acc
            return c2

        lax.fori_loop(0, NV, pass2, 0)
        pltpu.sync_copy(out_v, out_hbm.at[pl.ds(start, CHUNK)])
        return carry

    lax.fori_loop(0, NC, chunk_body, 0)


def kernel(img, transfos):
    img_flat = img.reshape(-1)
    imgpad = jnp.concatenate([img_flat, jnp.zeros((PAD,), jnp.float32)])
    img2 = _build(imgpad)
    ix, iy, iz = _coords(transfos)
    out = _interp(img2, ix, iy, iz)
    return out.reshape(img.shape)


# speedup vs baseline: 2.6345x; 1.1597x over previous
import functools

import jax
import jax.numpy as jnp
from jax import lax
from jax.experimental import pallas as pl
from jax.experimental.pallas import tpu as pltpu
from jax.experimental.pallas import tpu_sc as plsc

B, D, H, W = 4, 96, 96, 96
N = D * H * W
TOT = B * N
NTILES = 32
SPT = TOT // NTILES
CHUNK = 1536
NV = CHUNK // 16
NC = SPT // CHUNK
FMAX = float(W - 1)
SHIFTS = (0, 1, W, W + 1, H * W, H * W + 1, H * W + W, H * W + W + 1)


def _coords(transfos):
    zs = jnp.linspace(-1.0, 1.0, D)
    ys = jnp.linspace(-1.0, 1.0, H)
    xs = jnp.linspace(-1.0, 1.0, W)
    mz, my, mx = jnp.meshgrid(zs, ys, xs, indexing="ij")
    grid = jnp.stack([mx.reshape(-1), my.reshape(-1), mz.reshape(-1),
                      jnp.ones(N, dtype=jnp.float32)], axis=0)
    points = jnp.einsum("bij,jn->bin", transfos, grid)
    coeff = jnp.float32((W - 1) / 2.0)
    ix = coeff * points[:, 0, :] + coeff
    iy = coeff * points[:, 1, :] + coeff
    iz = coeff * points[:, 2, :] + coeff
    return ix.reshape(-1), iy.reshape(-1), iz.reshape(-1)


PAD = 9320
HHW = 1536 + PAD


@functools.partial(
    pl.kernel,
    out_type=jax.ShapeDtypeStruct((TOT, 8), jnp.float32),
    mesh=plsc.VectorSubcoreMesh(core_axis_name="c", subcore_axis_name="s"),
    compiler_params=pltpu.CompilerParams(needs_layout_passes=False,
                                         use_tc_tiling_on_sc=False),
    scratch_types=[
        pltpu.VMEM((HHW,), jnp.float32),
        pltpu.VMEM((1536, 8), jnp.float32),
    ],
)
def _build(imgpad_hbm, img2_hbm, win_v, buf_v):
    wid = lax.axis_index("s") * 2 + lax.axis_index("c")
    lanes = lax.iota(jnp.int32, 16)

    def chunk_body(g, carry):
        s = wid * SPT + g * 1536
        pltpu.sync_copy(imgpad_hbm.at[pl.ds(s, HHW)], win_v)

        def rows16(v, c2):
            base = v * 16
            rows = lanes + base
            for j, sh in enumerate(SHIFTS):
                vec = plsc.load_gather(win_v, [rows + sh])
                plsc.store_scatter(buf_v, [rows, jnp.full((16,), j, jnp.int32)], vec)
            return c2

        lax.fori_loop(0, NV, rows16, 0)
        pltpu.sync_copy(buf_v, img2_hbm.at[pl.ds(s, 1536)])
        return carry

    lax.fori_loop(0, NC, chunk_body, 0)


@functools.partial(
    pl.kernel,
    out_type=jax.ShapeDtypeStruct((TOT,), jnp.float32),
    mesh=plsc.VectorSubcoreMesh(core_axis_name="c", subcore_axis_name="s"),
    compiler_params=pltpu.CompilerParams(needs_layout_passes=False,
                                         use_tc_tiling_on_sc=False),
    scratch_types=[
        pltpu.VMEM((3 * CHUNK,), jnp.float32),
        pltpu.VMEM((3 * CHUNK,), jnp.float32),
        pltpu.VMEM((CHUNK,), jnp.int32),
        pltpu.VMEM((CHUNK,), jnp.int32),
        pltpu.VMEM((8 * CHUNK,), jnp.float32),
        pltpu.VMEM((8 * CHUNK,), jnp.float32),
        pltpu.VMEM((CHUNK, 8), jnp.float32),
        pltpu.VMEM((CHUNK, 8), jnp.float32),
        pltpu.VMEM((CHUNK,), jnp.float32),
        pltpu.SemaphoreType.DMA,
        pltpu.SemaphoreType.DMA,
    ],
)
def _interp(img2_hbm, ix_hbm, iy_hbm, iz_hbm, out_hbm,
            crd0, crd1, idx0, idx1, w0, w1, vals0, vals1, out_v, sem0, sem1):
    wid = lax.axis_index("s") * 2 + lax.axis_index("c")
    bbase = (wid // (NTILES // B)) * N
    bbv = jnp.full((16,), bbase, jnp.int32)
    lanes = lax.iota(jnp.int32, 16)

    def pass1(g, crd_v, idx_v, w_v):
        start = wid * SPT + g * CHUNK
        pltpu.sync_copy(ix_hbm.at[pl.ds(start, CHUNK)], crd_v.at[pl.ds(0, CHUNK)])
        pltpu.sync_copy(iy_hbm.at[pl.ds(start, CHUNK)], crd_v.at[pl.ds(CHUNK, CHUNK)])
        pltpu.sync_copy(iz_hbm.at[pl.ds(start, CHUNK)], crd_v.at[pl.ds(2 * CHUNK, CHUNK)])

        def body(v, c2):
            base = v * 16
            ix = crd_v[pl.ds(base, 16)]
            iy = crd_v[pl.ds(CHUNK + base, 16)]
            iz = crd_v[pl.ds(2 * CHUNK + base, 16)]
            valid = ((ix >= 0.0) & (ix <= FMAX) & (iy >= 0.0) & (iy <= FMAX)
                     & (iz >= 0.0) & (iz <= FMAX))
            vf = jnp.where(valid, jnp.float32(1.0), jnp.float32(0.0))
            ix0 = jnp.clip(ix, 0.0, FMAX).astype(jnp.int32)
            iy0 = jnp.clip(iy, 0.0, FMAX).astype(jnp.int32)
            iz0 = jnp.clip(iz, 0.0, FMAX).astype(jnp.int32)
            fx = ix - ix0.astype(jnp.float32)
            fy = iy - iy0.astype(jnp.float32)
            fz = iz - iz0.astype(jnp.float32)
            ux = (jnp.abs(fx - 1.0) * vf, jnp.abs(fx) * vf)
            uy = (jnp.abs(fy - 1.0), jnp.abs(fy))
            uz = (jnp.abs(fz - 1.0), jnp.abs(fz))
            uyz = (uy[0] * uz[0], uy[0] * uz[1], uy[1] * uz[0], uy[1] * uz[1])
            idx_v[pl.ds(base, 16)] = (ix0 + iy0 * W) + (iz0 * (H * W) + bbv)
            for c in range(8):
                dx, dy, dz = c >> 2, (c >> 1) & 1, c & 1
                w_v[pl.ds(c * CHUNK + base, 16)] = ux[dx] * uyz[2 * dy + dz]
            return c2

        lax.fori_loop(0, NV, body, 0)

    def pass2(g, w_v, vals_v):
        def body(v, c2):
            base = v * 16
            rows = lanes + base
            acc = None
            for c in range(8):
                dx, dy, dz = c >> 2, (c >> 1) & 1, c & 1
                col = jnp.full((16,), (dz << 2) | (dy << 1) | dx, jnp.int32)
                vc = plsc.load_gather(vals_v, [rows, col])
                wv = w_v[pl.ds(c * CHUNK + base, 16)]
                acc = wv * vc if acc is None else acc + wv * vc
            out_v[pl.ds(base, 16)] = acc
            return c2

        lax.fori_loop(0, NV, body, 0)
        pltpu.sync_copy(out_v, out_hbm.at[pl.ds(wid * SPT + g * CHUNK, CHUNK)])

    pass1(0, crd0, idx0, w0)
    pltpu.async_copy(img2_hbm.at[idx0], vals0, sem0)

    def two_chunks(h, carry):
        g0 = 2 * h
        pass1(g0 + 1, crd1, idx1, w1)
        pltpu.async_copy(img2_hbm.at[idx1], vals1, sem1)
        pltpu.make_async_copy(img2_hbm.at[idx0], vals0, sem0).wait()
        pass2(g0, w0, vals0)

        @pl.when(g0 + 2 < NC)
        def _():
            pass1(g0 + 2, crd0, idx0, w0)
            pltpu.async_copy(img2_hbm.at[idx0], vals0, sem0)

        pltpu.make_async_copy(img2_hbm.at[idx1], vals1, sem1).wait()
        pass2(g0 + 1, w1, vals1)
        return carry

    lax.fori_loop(0, NC // 2, two_chunks, 0)


def kernel(img, transfos):
    img_flat = img.reshape(-1)
    imgpad = jnp.concatenate([img_flat, jnp.zeros((PAD,), jnp.float32)])
    img2 = _build(imgpad)
    ix, iy, iz = _coords(transfos)
    out = _interp(img2, ix, iy, iz)
    return out.reshape(img.shape)
